# baseline (speedup 1.0000x reference)
import jax
import jax.numpy as jnp
from jax import lax
from jax.experimental import pallas as pl
from jax.experimental.pallas import tpu as pltpu
from jax.experimental.pallas import tpu_sc as plsc

_K = 256
_B = 128
_N = 32768
_L = 16
_NC = 2
_NS = 16
_NW = _NC * _NS
_RPW = _B // _NW
_NCHUNK = _N // _L
_NBUCK = 256
_NV = _K // _L
_MIN_I32 = -2147483648


def _lanes():
    return lax.iota(jnp.int32, _L)


def _pcnt(mask):
    return plsc.all_reduce_population_count(mask)


def _fill_zero(ref, nwords):
    zero = jnp.zeros((_L,), jnp.int32)

    def body(i):
        ref[pl.ds(i * _L, _L)] = zero

    plsc.parallel_loop(0, nwords // _L, unroll=8)(body)


def _scan_down(hist, need_v):
    zero16 = jnp.zeros((_L,), jnp.int32)

    def body(i, carry):
        cum_v, fb_v = carry
        c = 15 - i
        base = c * _L
        parts = [hist[pl.ds(l * _NBUCK + base, _L)] for l in range(_L)]
        while len(parts) > 1:
            parts = [parts[k] + parts[k + 1]
                     for k in range(0, len(parts), 2)]
        rev = lax.rev(parts[0], (0,))
        cs = plsc.cumsum(rev)
        tot_v = jnp.full((_L,), cs[_L - 1], jnp.int32)
        hit = (cum_v + cs) >= need_v
        p_v = plsc.all_reduce_ffs(hit)
        any_v = _pcnt(hit) > 0
        b_here_v = (base + (_L - 1)) - p_v
        take = jnp.logical_and(fb_v < 0, any_v)
        fb_v = jnp.where(take, b_here_v, fb_v)
        return cum_v + tot_v, fb_v

    _, fb_v = plsc.parallel_loop(
        0, 16, carry=(zero16, zero16 - 1))(body)
    return fb_v


def _sc_body(x_hbm, mask_hbm, idx_hbm,
             krow, mrow, cand, hist_a, hist_b, sel, selkey, skey2, outidx,
             sem_in, sem_mout, sem_iout):
    lanes = _lanes()
    lane_base = lanes * jnp.int32(_NBUCK)
    ones = jnp.ones((_L,), jnp.int32)
    neg_ones = -ones
    zero16 = jnp.zeros((_L,), jnp.int32)
    k_v = jnp.full((_L,), _K, jnp.int32)

    cid = lax.axis_index("c")
    sid = lax.axis_index("s")
    wid = sid * _NC + cid
    row0 = wid * _RPW

    _fill_zero(hist_b, _NBUCK * _L)
    pltpu.async_copy(x_hbm.at[row0], krow, sem_in)

    def row_body(r, rcarry):
        row = row0 + r
        _fill_zero(hist_a, _NBUCK * _L)
        pltpu.make_async_copy(x_hbm.at[row], krow, sem_in).wait()

        def phase1(i):
            xb = krow[pl.ds(i * _L, _L)]
            b = plsc.bitcast(xb, jnp.int32)
            ku = b ^ ((b >> 31) | jnp.int32(_MIN_I32))
            krow[pl.ds(i * _L, _L)] = plsc.bitcast(ku, jnp.float32)
            kuu = plsc.bitcast(ku, jnp.uint32)
            bucket = plsc.bitcast(kuu >> 24, jnp.int32)
            plsc.addupdate_scatter(hist_a, [lane_base + bucket], ones)

        plsc.parallel_loop(0, _NCHUNK, unroll=8)(phase1)

        b1_v = _scan_down(hist_a, k_v)

        @pl.when(r > 0)
        def _():
            pltpu.make_async_copy(mrow, mask_hbm.at[row], sem_mout).wait()
            pltpu.make_async_copy(outidx, idx_hbm.at[row], sem_iout).wait()

        def phase2(i, coff_v):
            kb = plsc.bitcast(krow[pl.ds(i * _L, _L)], jnp.uint32)
            bucket = plsc.bitcast(kb >> 24, jnp.int32)
            iv = i * _L + lanes
            mgt = bucket > b1_v
            mge = bucket >= b1_v
            mrow[pl.ds(i * _L, _L)] = jnp.where(mgt, ones, zero16)
            plsc.store_compressed(cand.at[pl.ds(coff_v[0], _L)], iv,
                                  mask=mge)
            return coff_v + _pcnt(mge)

        coff_v = plsc.parallel_loop(0, _NCHUNK, unroll=4,
                                    carry=zero16)(phase2)
        m = coff_v[0]

        def fsplit(j, carry):
            soff_v, koff_v = carry
            pos = j * _L + lanes
            valid = pos < coff_v
            iv = cand[pl.ds(j * _L, _L)]
            kv = plsc.load_gather(krow, [iv], mask=valid)
            kb = plsc.bitcast(kv, jnp.uint32)
            bucket = plsc.bitcast(kb >> 24, jnp.int32)
            mwin = jnp.logical_and(bucket > b1_v, valid)
            mkeep = jnp.logical_and(bucket == b1_v, valid)
            plsc.store_compressed(sel.at[pl.ds(soff_v[0], _L)], iv,
                                  mask=mwin)
            plsc.store_compressed(cand.at[pl.ds(koff_v[0], _L)], iv,
                                  mask=mkeep)
            return soff_v + _pcnt(mwin), koff_v + _pcnt(mkeep)

        soff_v, koff_v = plsc.parallel_loop(
            0, (m + _L - 1) // _L, unroll=2, carry=(zero16, zero16))(fsplit)
        m_v = koff_v
        m = m_v[0]
        need_v = k_v - soff_v

        for lvl in (2, 3, 4):
            shift = jnp.uint32(32 - 8 * lvl)
            nchunks = (m + _L - 1) // _L

            def fhist(j, m_v=m_v, shift=shift):
                pos = j * _L + lanes
                valid = pos < m_v
                iv = cand[pl.ds(j * _L, _L)]
                kv = plsc.load_gather(krow, [iv], mask=valid)
                kb = plsc.bitcast(kv, jnp.uint32)
                byte = plsc.bitcast((kb >> shift) & jnp.uint32(0xFF),
                                    jnp.int32)
                plsc.addupdate_scatter(hist_b, [lane_base + byte], ones,
                                       mask=valid)

            plsc.parallel_loop(0, nchunks, unroll=2)(fhist)
            bl_v = _scan_down(hist_b, need_v)

            def fcomp(j, carry, m_v=m_v, shift=shift, bl_v=bl_v):
                soff_v, koff_v = carry
                pos = j * _L + lanes
                valid = pos < m_v
                iv = cand[pl.ds(j * _L, _L)]
                kv = plsc.load_gather(krow, [iv], mask=valid)
                kb = plsc.bitcast(kv, jnp.uint32)
                byte = plsc.bitcast((kb >> shift) & jnp.uint32(0xFF),
                                    jnp.int32)
                plsc.addupdate_scatter(hist_b, [lane_base + byte], neg_ones,
                                       mask=valid)
                mwin = jnp.logical_and(byte > bl_v, valid)
                mkeep = jnp.logical_and(byte == bl_v, valid)
                plsc.store_compressed(sel.at[pl.ds(soff_v[0], _L)], iv,
                                      mask=mwin)
                plsc.store_compressed(cand.at[pl.ds(koff_v[0], _L)], iv,
                                      mask=mkeep)
                return soff_v + _pcnt(mwin), koff_v + _pcnt(mkeep)

            soff_v, koff_v = plsc.parallel_loop(
                0, nchunks, carry=(soff_v, zero16))(fcomp)
            m_v = koff_v
            m = m_v[0]
            need_v = k_v - soff_v

        tie_base = soff_v[0]

        def ftie(j):
            pos = j * _L + lanes
            valid = pos < need_v
            iv = cand[pl.ds(j * _L, _L)]
            plsc.store_compressed(sel.at[pl.ds(tie_base + j * _L, _L)], iv,
                                  mask=valid)

        plsc.parallel_loop(0, (need_v[0] + _L - 1) // _L)(ftie)

        def fmsc(jb):
            iv = sel[pl.ds(jb * _L, _L)]
            plsc.store_scatter(mrow, [iv], ones)

        plsc.parallel_loop(0, _NV)(fmsc)
        pltpu.async_copy(mrow, mask_hbm.at[row], sem_mout)

        def fgath(jb):
            ivb = sel[pl.ds(jb * _L, _L)]
            kvb = plsc.load_gather(krow, [ivb])
            selkey[pl.ds(jb * _L, _L)] = plsc.bitcast(kvb, jnp.int32)

        plsc.parallel_loop(0, _NV)(fgath)

        @pl.when(r + 1 < _RPW)
        def _():
            pltpu.async_copy(x_hbm.at[row + 1], krow, sem_in)

        bk = [plsc.bitcast(selkey[pl.ds(v * _L, _L)], jnp.uint32)
              for v in range(_NV)]
        bv = [lanes + v * _L for v in range(_NV)]
        for v in range(_NV):
            bk[v], bv[v] = plsc.sort_key_val(bk[v], bv[v],
                                             descending=(v & 1) == 1)
        for kk in (32, 64, 128, 256):
            j = kk // 2
            while j >= _L:
                vj = j // _L
                for v in range(_NV):
                    p = v ^ vj
                    if p > v:
                        up = ((v * _L) & kk) == 0
                        ka, kb, va, vb = bk[v], bk[p], bv[v], bv[p]
                        mle = (ka <= kb) if up else (kb <= ka)
                        bk[v] = jnp.where(mle, ka, kb)
                        bk[p] = jnp.where(mle, kb, ka)
                        bv[v] = jnp.where(mle, va, vb)
                        bv[p] = jnp.where(mle, vb, va)
                j //= 2
            for v in range(_NV):
                up = ((v * _L) & kk) == 0
                bk[v], bv[v] = plsc.sort_key_val(bk[v], bv[v],
                                                 descending=not up)

        skey2[pl.ds(_K, _L)] = zero16
        for v in range(_NV):
            skey2[pl.ds(v * _L, _L)] = plsc.bitcast(bk[v], jnp.int32)
        tiem = skey2[pl.ds(0, _L)] == skey2[pl.ds(1, _L)]
        for v in range(1, _NV):
            a = skey2[pl.ds(v * _L, _L)] == skey2[pl.ds(v * _L + 1, _L)]
            tiem = jnp.logical_or(tiem, a)
        prev = skey2[pl.ds(_L - 1, _L)] == skey2[pl.ds(_L, _L)]
        tiem = jnp.logical_or(tiem, prev)
        any_tie = _pcnt(tiem)[0] > 0

        def fast_path():
            for v in range(_NV):
                rank = (_K - 1 - v * _L) - lanes
                orig = plsc.load_gather(sel, [bv[v]])
                plsc.store_scatter(outidx, [rank], orig)

        def tie_path():
            def frank(ib, c):
                vkey = plsc.bitcast(selkey[pl.ds(ib * _L, _L)], jnp.uint32)

                def fge(jb, acc):
                    kj16 = selkey[pl.ds(jb * _L, _L)]
                    for l in range(_L):
                        kjs = plsc.bitcast(
                            jnp.full((_L,), kj16[l], jnp.int32), jnp.uint32)
                        acc = acc + jnp.where(kjs >= vkey, ones, zero16)
                    return acc

                def fgt(jb, acc):
                    kj16 = selkey[pl.ds(jb * _L, _L)]
                    for l in range(_L):
                        kjs = plsc.bitcast(
                            jnp.full((_L,), kj16[l], jnp.int32), jnp.uint32)
                        acc = acc + jnp.where(kjs > vkey, ones, zero16)
                    return acc

                acc = lax.fori_loop(0, ib, fge, zero16)
                acc = lax.fori_loop(ib + 1, _NV, fgt, acc)
                kd16 = selkey[pl.ds(ib * _L, _L)]
                for l in range(_L):
                    kjs = plsc.bitcast(
                        jnp.full((_L,), kd16[l], jnp.int32), jnp.uint32)
                    hit = jnp.where(l < lanes, kjs >= vkey, kjs > vkey)
                    acc = acc + jnp.where(hit, ones, zero16)
                plsc.store_scatter(outidx, [acc], sel[pl.ds(ib * _L, _L)])
                return c

            lax.fori_loop(0, _NV, frank, 0)

        lax.cond(any_tie, tie_path, fast_path)

        pltpu.async_copy(outidx, idx_hbm.at[row], sem_iout)
        return rcarry

    lax.fori_loop(0, _RPW, row_body, 0)
    last = row0 + _RPW - 1
    pltpu.make_async_copy(mrow, mask_hbm.at[last], sem_mout).wait()
    pltpu.make_async_copy(outidx, idx_hbm.at[last], sem_iout).wait()


_topk_call = pl.kernel(
    _sc_body,
    out_type=(
        jax.ShapeDtypeStruct((_B, _N), jnp.int32),
        jax.ShapeDtypeStruct((_B, _K), jnp.int32),
    ),
    mesh=plsc.VectorSubcoreMesh(
        core_axis_name="c", subcore_axis_name="s",
        num_cores=_NC, num_subcores=_NS),
    compiler_params=pltpu.CompilerParams(needs_layout_passes=False),
    scratch_types=[
        pltpu.VMEM((_N,), jnp.float32),
        pltpu.VMEM((_N,), jnp.int32),
        pltpu.VMEM((_N + _L,), jnp.int32),
        pltpu.VMEM((_NBUCK * _L,), jnp.int32),
        pltpu.VMEM((_NBUCK * _L,), jnp.int32),
        pltpu.VMEM((_K + _L,), jnp.int32),
        pltpu.VMEM((_K + _L,), jnp.int32),
        pltpu.VMEM((_K + _L,), jnp.int32),
        pltpu.VMEM((_K,), jnp.int32),
        pltpu.SemaphoreType.DMA,
        pltpu.SemaphoreType.DMA,
        pltpu.SemaphoreType.DMA,
    ],
)


def kernel(x):
    mask_i32, idx = _topk_call(x)
    return mask_i32.astype(jnp.bool_), idx


# speedup vs baseline: 1.0067x; 1.0067x over previous
import jax
import jax.numpy as jnp
from jax import lax
from jax.experimental import pallas as pl
from jax.experimental.pallas import tpu as pltpu
from jax.experimental.pallas import tpu_sc as plsc

_K = 256
_B = 128
_N = 32768
_L = 16
_NC = 2
_NS = 16
_NW = _NC * _NS
_RPW = _B // _NW
_NCHUNK = _N // _L
_NBUCK = 256
_NV = _K // _L
_MIN_I32 = -2147483648


def _lanes():
    return lax.iota(jnp.int32, _L)


def _pcnt(mask):
    return plsc.all_reduce_population_count(mask)


def _fill_zero(ref, nwords):
    zero = jnp.zeros((_L,), jnp.int32)

    def body(i):
        ref[pl.ds(i * _L, _L)] = zero

    plsc.parallel_loop(0, nwords // _L, unroll=8)(body)


def _scan_down(hist, need_v):
    zero16 = jnp.zeros((_L,), jnp.int32)

    def body(i, carry):
        cum_v, fb_v = carry
        c = 15 - i
        base = c * _L
        parts = [hist[pl.ds(l * _NBUCK + base, _L)] for l in range(_L)]
        while len(parts) > 1:
            parts = [parts[k] + parts[k + 1]
                     for k in range(0, len(parts), 2)]
        rev = lax.rev(parts[0], (0,))
        cs = plsc.cumsum(rev)
        tot_v = jnp.full((_L,), cs[_L - 1], jnp.int32)
        hit = (cum_v + cs) >= need_v
        p_v = plsc.all_reduce_ffs(hit)
        any_v = _pcnt(hit) > 0
        b_here_v = (base + (_L - 1)) - p_v
        take = jnp.logical_and(fb_v < 0, any_v)
        fb_v = jnp.where(take, b_here_v, fb_v)
        return cum_v + tot_v, fb_v

    _, fb_v = plsc.parallel_loop(
        0, 16, carry=(zero16, zero16 - 1))(body)
    return fb_v


def _sc_body(x_hbm, mask_hbm, idx_hbm,
             krow, mrow, cand, hist_a, hist_b, sel, selkey, skey2, outidx,
             sem_in, sem_mout, sem_iout):
    lanes = _lanes()
    lane_base = lanes * jnp.int32(_NBUCK)
    ones = jnp.ones((_L,), jnp.int32)
    neg_ones = -ones
    zero16 = jnp.zeros((_L,), jnp.int32)
    k_v = jnp.full((_L,), _K, jnp.int32)

    cid = lax.axis_index("c")
    sid = lax.axis_index("s")
    wid = sid * _NC + cid
    row0 = wid * _RPW

    _fill_zero(hist_b, _NBUCK * _L)
    pltpu.async_copy(x_hbm.at[row0], krow, sem_in)

    def row_body(r, rcarry):
        row = row0 + r
        _fill_zero(hist_a, _NBUCK * _L)
        pltpu.make_async_copy(x_hbm.at[row], krow, sem_in).wait()

        def phase1(i):
            xb = krow[pl.ds(i * _L, _L)]
            b = plsc.bitcast(xb, jnp.int32)
            ku = b ^ ((b >> 31) | jnp.int32(_MIN_I32))
            krow[pl.ds(i * _L, _L)] = plsc.bitcast(ku, jnp.float32)
            kuu = plsc.bitcast(ku, jnp.uint32)
            bucket = plsc.bitcast(kuu >> 24, jnp.int32)
            plsc.addupdate_scatter(hist_a, [lane_base + bucket], ones)

        plsc.parallel_loop(0, _NCHUNK, unroll=4)(phase1)

        b1_v = _scan_down(hist_a, k_v)

        @pl.when(r > 0)
        def _():
            pltpu.make_async_copy(mrow, mask_hbm.at[row], sem_mout).wait()
            pltpu.make_async_copy(outidx, idx_hbm.at[row], sem_iout).wait()

        def phase2(i, coff_v):
            kb = plsc.bitcast(krow[pl.ds(i * _L, _L)], jnp.uint32)
            bucket = plsc.bitcast(kb >> 24, jnp.int32)
            iv = i * _L + lanes
            mgt = bucket > b1_v
            mge = bucket >= b1_v
            mrow[pl.ds(i * _L, _L)] = jnp.where(mgt, ones, zero16)
            plsc.store_compressed(cand.at[pl.ds(coff_v[0], _L)], iv,
                                  mask=mge)
            return coff_v + _pcnt(mge)

        coff_v = plsc.parallel_loop(0, _NCHUNK, unroll=4,
                                    carry=zero16)(phase2)
        m = coff_v[0]

        def fsplit(j, carry):
            soff_v, koff_v = carry
            pos = j * _L + lanes
            valid = pos < coff_v
            iv = cand[pl.ds(j * _L, _L)]
            kv = plsc.load_gather(krow, [iv], mask=valid)
            kb = plsc.bitcast(kv, jnp.uint32)
            bucket = plsc.bitcast(kb >> 24, jnp.int32)
            mwin = jnp.logical_and(bucket > b1_v, valid)
            mkeep = jnp.logical_and(bucket == b1_v, valid)
            plsc.store_compressed(sel.at[pl.ds(soff_v[0], _L)], iv,
                                  mask=mwin)
            plsc.store_compressed(cand.at[pl.ds(koff_v[0], _L)], iv,
                                  mask=mkeep)
            return soff_v + _pcnt(mwin), koff_v + _pcnt(mkeep)

        soff_v, koff_v = plsc.parallel_loop(
            0, (m + _L - 1) // _L, carry=(zero16, zero16))(fsplit)
        m_v = koff_v
        m = m_v[0]
        need_v = k_v - soff_v

        for lvl in (2, 3, 4):
            shift = jnp.uint32(32 - 8 * lvl)
            nchunks = (m + _L - 1) // _L

            def fhist(j, m_v=m_v, shift=shift):
                pos = j * _L + lanes
                valid = pos < m_v
                iv = cand[pl.ds(j * _L, _L)]
                kv = plsc.load_gather(krow, [iv], mask=valid)
                kb = plsc.bitcast(kv, jnp.uint32)
                byte = plsc.bitcast((kb >> shift) & jnp.uint32(0xFF),
                                    jnp.int32)
                plsc.addupdate_scatter(hist_b, [lane_base + byte], ones,
                                       mask=valid)

            plsc.parallel_loop(0, nchunks)(fhist)
            bl_v = _scan_down(hist_b, need_v)

            def fcomp(j, carry, m_v=m_v, shift=shift, bl_v=bl_v):
                soff_v, koff_v = carry
                pos = j * _L + lanes
                valid = pos < m_v
                iv = cand[pl.ds(j * _L, _L)]
                kv = plsc.load_gather(krow, [iv], mask=valid)
                kb = plsc.bitcast(kv, jnp.uint32)
                byte = plsc.bitcast((kb >> shift) & jnp.uint32(0xFF),
                                    jnp.int32)
                plsc.addupdate_scatter(hist_b, [lane_base + byte], neg_ones,
                                       mask=valid)
                mwin = jnp.logical_and(byte > bl_v, valid)
                mkeep = jnp.logical_and(byte == bl_v, valid)
                plsc.store_compressed(sel.at[pl.ds(soff_v[0], _L)], iv,
                                      mask=mwin)
                plsc.store_compressed(cand.at[pl.ds(koff_v[0], _L)], iv,
                                      mask=mkeep)
                return soff_v + _pcnt(mwin), koff_v + _pcnt(mkeep)

            soff_v, koff_v = plsc.parallel_loop(
                0, nchunks, carry=(soff_v, zero16))(fcomp)
            m_v = koff_v
            m = m_v[0]
            need_v = k_v - soff_v

        tie_base = soff_v[0]

        def ftie(j):
            pos = j * _L + lanes
            valid = pos < need_v
            iv = cand[pl.ds(j * _L, _L)]
            plsc.store_compressed(sel.at[pl.ds(tie_base + j * _L, _L)], iv,
                                  mask=valid)

        plsc.parallel_loop(0, (need_v[0] + _L - 1) // _L)(ftie)

        def fmsc(jb):
            iv = sel[pl.ds(jb * _L, _L)]
            plsc.store_scatter(mrow, [iv], ones)

        plsc.parallel_loop(0, _NV)(fmsc)
        pltpu.async_copy(mrow, mask_hbm.at[row], sem_mout)

        def fgath(jb):
            ivb = sel[pl.ds(jb * _L, _L)]
            kvb = plsc.load_gather(krow, [ivb])
            selkey[pl.ds(jb * _L, _L)] = plsc.bitcast(kvb, jnp.int32)

        plsc.parallel_loop(0, _NV)(fgath)

        @pl.when(r + 1 < _RPW)
        def _():
            pltpu.async_copy(x_hbm.at[row + 1], krow, sem_in)

        bk = [plsc.bitcast(selkey[pl.ds(v * _L, _L)], jnp.uint32)
              for v in range(_NV)]
        bv = [lanes + v * _L for v in range(_NV)]
        for v in range(_NV):
            bk[v], bv[v] = plsc.sort_key_val(bk[v], bv[v],
                                             descending=(v & 1) == 1)
        for kk in (32, 64, 128, 256):
            j = kk // 2
            while j >= _L:
                vj = j // _L
                for v in range(_NV):
                    p = v ^ vj
                    if p > v:
                        up = ((v * _L) & kk) == 0
                        ka, kb, va, vb = bk[v], bk[p], bv[v], bv[p]
                        mle = (ka <= kb) if up else (kb <= ka)
                        bk[v] = jnp.where(mle, ka, kb)
                        bk[p] = jnp.where(mle, kb, ka)
                        bv[v] = jnp.where(mle, va, vb)
                        bv[p] = jnp.where(mle, vb, va)
                j //= 2
            for v in range(_NV):
                up = ((v * _L) & kk) == 0
                bk[v], bv[v] = plsc.sort_key_val(bk[v], bv[v],
                                                 descending=not up)

        skey2[pl.ds(_K, _L)] = zero16
        for v in range(_NV):
            skey2[pl.ds(v * _L, _L)] = plsc.bitcast(bk[v], jnp.int32)
        tiem = skey2[pl.ds(0, _L)] == skey2[pl.ds(1, _L)]
        for v in range(1, _NV):
            a = skey2[pl.ds(v * _L, _L)] == skey2[pl.ds(v * _L + 1, _L)]
            tiem = jnp.logical_or(tiem, a)
        prev = skey2[pl.ds(_L - 1, _L)] == skey2[pl.ds(_L, _L)]
        tiem = jnp.logical_or(tiem, prev)
        any_tie = _pcnt(tiem)[0] > 0

        def fast_path():
            for v in range(_NV):
                rank = (_K - 1 - v * _L) - lanes
                orig = plsc.load_gather(sel, [bv[v]])
                plsc.store_scatter(outidx, [rank], orig)

        def tie_path():
            def frank(ib, c):
                vkey = plsc.bitcast(selkey[pl.ds(ib * _L, _L)], jnp.uint32)

                def fge(jb, acc):
                    kj16 = selkey[pl.ds(jb * _L, _L)]
                    for l in range(_L):
                        kjs = plsc.bitcast(
                            jnp.full((_L,), kj16[l], jnp.int32), jnp.uint32)
                        acc = acc + jnp.where(kjs >= vkey, ones, zero16)
                    return acc

                def fgt(jb, acc):
                    kj16 = selkey[pl.ds(jb * _L, _L)]
                    for l in range(_L):
                        kjs = plsc.bitcast(
                            jnp.full((_L,), kj16[l], jnp.int32), jnp.uint32)
                        acc = acc + jnp.where(kjs > vkey, ones, zero16)
                    return acc

                acc = lax.fori_loop(0, ib, fge, zero16)
                acc = lax.fori_loop(ib + 1, _NV, fgt, acc)
                kd16 = selkey[pl.ds(ib * _L, _L)]
                for l in range(_L):
                    kjs = plsc.bitcast(
                        jnp.full((_L,), kd16[l], jnp.int32), jnp.uint32)
                    hit = jnp.where(l < lanes, kjs >= vkey, kjs > vkey)
                    acc = acc + jnp.where(hit, ones, zero16)
                plsc.store_scatter(outidx, [acc], sel[pl.ds(ib * _L, _L)])
                return c

            lax.fori_loop(0, _NV, frank, 0)

        lax.cond(any_tie, tie_path, fast_path)

        pltpu.async_copy(outidx, idx_hbm.at[row], sem_iout)
        return rcarry

    lax.fori_loop(0, _RPW, row_body, 0)
    last = row0 + _RPW - 1
    pltpu.make_async_copy(mrow, mask_hbm.at[last], sem_mout).wait()
    pltpu.make_async_copy(outidx, idx_hbm.at[last], sem_iout).wait()


_topk_call = pl.kernel(
    _sc_body,
    out_type=(
        jax.ShapeDtypeStruct((_B, _N), jnp.int32),
        jax.ShapeDtypeStruct((_B, _K), jnp.int32),
    ),
    mesh=plsc.VectorSubcoreMesh(
        core_axis_name="c", subcore_axis_name="s",
        num_cores=_NC, num_subcores=_NS),
    compiler_params=pltpu.CompilerParams(needs_layout_passes=False),
    scratch_types=[
        pltpu.VMEM((_N,), jnp.float32),
        pltpu.VMEM((_N,), jnp.int32),
        pltpu.VMEM((_N + _L,), jnp.int32),
        pltpu.VMEM((_NBUCK * _L,), jnp.int32),
        pltpu.VMEM((_NBUCK * _L,), jnp.int32),
        pltpu.VMEM((_K + _L,), jnp.int32),
        pltpu.VMEM((_K + _L,), jnp.int32),
        pltpu.VMEM((_K + _L,), jnp.int32),
        pltpu.VMEM((_K,), jnp.int32),
        pltpu.SemaphoreType.DMA,
        pltpu.SemaphoreType.DMA,
        pltpu.SemaphoreType.DMA,
    ],
)


def kernel(x):
    mask_i32, idx = _topk_call(x)
    return mask_i32.astype(jnp.bool_), idx
